# Initial kernel scaffold; baseline (speedup 1.0000x reference)
import jax
import jax.numpy as jnp
from jax.experimental import pallas as pl


def kernel(x, edge_index, W1l, W1r, b1, W2l, W2r, b2):
    raise NotImplementedError("write your pallas kernel here")


# speedup vs baseline: 13.5928x; 13.5928x over previous
import functools

import jax
import jax.numpy as jnp
from jax import lax
from jax.experimental import pallas as pl
from jax.experimental.pallas import tpu as pltpu
from jax.experimental.pallas import tpu_sc as plsc

N = 10000
F_IN = 128
H = 16
C = 40
E = 320000

NC = 2
NS = 16
LANES = 16
NW = NC * NS

BLK = 128
NBLK = E // BLK
RB = 16
NCHUNK = -(-NBLK // (NW * RB))
BPW = RB * NCHUNK
NBLK_PAD = BPW * NW
E_PAD = NBLK_PAD * BLK
NPAD = 10240
RPT = NPAD // NS


def _sc_mesh():
    return plsc.VectorSubcoreMesh(
        core_axis_name="c", subcore_axis_name="s",
        num_cores=NC, num_subcores=NS)


def _make_agg(with_deg):
    out_type = [jax.ShapeDtypeStruct((NC, NPAD, H), jnp.float32)]
    scratch = [
        pltpu.VMEM((RB, BLK), jnp.int32),
        pltpu.VMEM((RB, BLK), jnp.int32),
        pltpu.VMEM((RB * BLK, H), jnp.float32),
        pltpu.VMEM((BLK, H), jnp.float32),
        pltpu.VMEM_SHARED((NPAD, H), jnp.float32),
        pltpu.SemaphoreType.DMA,
    ]
    if with_deg:
        out_type.append(jax.ShapeDtypeStruct((NC, NPAD), jnp.float32))
        scratch += [
            pltpu.VMEM((BLK,), jnp.float32),
            pltpu.VMEM((RPT,), jnp.float32),
            pltpu.VMEM_SHARED((NPAD,), jnp.float32),
        ]

    def body(table, srcb, dstb, *rest):
        if with_deg:
            (out_acc, out_deg, srcv, dstv, rows, zrow, acc_sh, sem,
             ones, zdeg, deg_sh) = rest
        else:
            out_acc, srcv, dstv, rows, zrow, acc_sh, sem = rest
        cid = lax.axis_index("c")
        sid = lax.axis_index("s")
        wid = cid * NS + sid
        zero16 = jnp.zeros((LANES,), jnp.float32)

        def zrow_body(i, carry):
            zrow[i, :] = zero16
            return carry
        lax.fori_loop(0, BLK, zrow_body, 0)

        row0 = pl.multiple_of(sid * RPT, RPT)
        for z in range(RPT // BLK):
            pltpu.sync_copy(zrow, acc_sh.at[pl.ds(row0 + z * BLK, BLK)])

        if with_deg:
            def zdeg_body(i, carry):
                zdeg[pl.ds(i * LANES, LANES)] = zero16
                return carry
            lax.fori_loop(0, RPT // LANES, zdeg_body, 0)

            one16 = jnp.ones((LANES,), jnp.float32)

            def ones_body(i, carry):
                ones[pl.ds(i * LANES, LANES)] = one16
                return carry
            lax.fori_loop(0, BLK // LANES, ones_body, 0)
            pltpu.sync_copy(zdeg, deg_sh.at[pl.ds(row0, RPT)])

        plsc.subcore_barrier()

        base_blk = wid * BPW

        def chunk(c, carry):
            blk0 = pl.multiple_of(base_blk + c * RB, RB)
            pltpu.sync_copy(srcb.at[pl.ds(blk0, RB)], srcv)
            pltpu.sync_copy(dstb.at[pl.ds(blk0, RB)], dstv)
            descs = [
                pltpu.async_copy(table.at[srcv.at[k]],
                                 rows.at[pl.ds(k * BLK, BLK)], sem)
                for k in range(RB)
            ]
            for d in descs:
                d.wait()
            for k in range(RB):
                pltpu.sync_copy(rows.at[pl.ds(k * BLK, BLK)],
                                acc_sh.at[dstv.at[k]], add=True)
                if with_deg:
                    pltpu.sync_copy(ones, deg_sh.at[dstv.at[k]], add=True)
            return carry
        lax.fori_loop(0, NCHUNK, chunk, 0)

        plsc.subcore_barrier()

        pltpu.sync_copy(acc_sh.at[pl.ds(row0, RPT)],
                        out_acc.at[cid, pl.ds(row0, RPT)])
        if with_deg:
            pltpu.sync_copy(deg_sh.at[pl.ds(row0, RPT)],
                            out_deg.at[cid, pl.ds(row0, RPT)])

    return pl.kernel(body, out_type=out_type, mesh=_sc_mesh(),
                     scratch_types=scratch,
                     compiler_params=pltpu.CompilerParams(
                         use_tc_tiling_on_sc=False))


_G = 5
_BM = N // _G


def _mm1(x, W1l, W1r, b1):
    def body(x_ref, wl_ref, wr_ref, b_ref, xl_ref, xr_ref):
        xv = x_ref[...]
        xl_ref[...] = jnp.dot(xv, wl_ref[...],
                              preferred_element_type=jnp.float32, precision=lax.Precision.HIGHEST)
        xr_ref[...] = jnp.dot(xv, wr_ref[...],
                              preferred_element_type=jnp.float32, precision=lax.Precision.HIGHEST) + b_ref[...]

    return pl.pallas_call(
        body,
        grid=(_G,),
        in_specs=[
            pl.BlockSpec((_BM, F_IN), lambda i: (i, 0)),
            pl.BlockSpec((F_IN, H), lambda i: (0, 0)),
            pl.BlockSpec((F_IN, H), lambda i: (0, 0)),
            pl.BlockSpec((1, H), lambda i: (0, 0)),
        ],
        out_specs=[
            pl.BlockSpec((_BM, H), lambda i: (i, 0)),
            pl.BlockSpec((_BM, H), lambda i: (i, 0)),
        ],
        out_shape=[
            jax.ShapeDtypeStruct((N, H), jnp.float32),
            jax.ShapeDtypeStruct((N, H), jnp.float32),
        ],
    )(x, W1l, W1r, b1)


def _mean_relu(p, deg2, xr):
    def body(p_ref, d_ref, xr_ref, h_ref):
        s = p_ref[0] + p_ref[1]
        r = 1.0 / jnp.maximum(d_ref[0] + d_ref[1], 1.0)
        h_ref[...] = jnp.maximum(s * r + xr_ref[...], 0.0)

    return pl.pallas_call(
        body,
        grid=(_G,),
        in_specs=[
            pl.BlockSpec((NC, _BM, H), lambda i: (0, i, 0)),
            pl.BlockSpec((NC, _BM, 1), lambda i: (0, i, 0)),
            pl.BlockSpec((_BM, H), lambda i: (i, 0)),
        ],
        out_specs=pl.BlockSpec((_BM, H), lambda i: (i, 0)),
        out_shape=jax.ShapeDtypeStruct((N, H), jnp.float32),
    )(p, deg2, xr)


def _final(q, deg2, h, W2l, W2r, b2):
    def body(q_ref, d_ref, h_ref, wl_ref, wr_ref, b_ref, o_ref):
        m = (q_ref[0] + q_ref[1]) * (
            1.0 / jnp.maximum(d_ref[0] + d_ref[1], 1.0))
        z = (jnp.dot(m, wl_ref[...], preferred_element_type=jnp.float32, precision=lax.Precision.HIGHEST)
             + jnp.dot(h_ref[...], wr_ref[...],
                       preferred_element_type=jnp.float32, precision=lax.Precision.HIGHEST)
             + b_ref[...])
        z = z - jnp.max(z, axis=1, keepdims=True)
        o_ref[...] = z - jnp.log(jnp.sum(jnp.exp(z), axis=1, keepdims=True))

    return pl.pallas_call(
        body,
        grid=(_G,),
        in_specs=[
            pl.BlockSpec((NC, _BM, H), lambda i: (0, i, 0)),
            pl.BlockSpec((NC, _BM, 1), lambda i: (0, i, 0)),
            pl.BlockSpec((_BM, H), lambda i: (i, 0)),
            pl.BlockSpec((H, C), lambda i: (0, 0)),
            pl.BlockSpec((H, C), lambda i: (0, 0)),
            pl.BlockSpec((1, C), lambda i: (0, 0)),
        ],
        out_specs=pl.BlockSpec((_BM, C), lambda i: (i, 0)),
        out_shape=jax.ShapeDtypeStruct((N, C), jnp.float32),
    )(q, deg2, h, W2l, W2r, b2)


def kernel(x, edge_index, W1l, W1r, b1, W2l, W2r, b2):
    src = edge_index[0]
    dst = edge_index[1]
    pad_e = E_PAD - E
    pad_src = jnp.zeros((pad_e,), jnp.int32)
    pad_dst = N + (jnp.arange(pad_e, dtype=jnp.int32) % (NPAD - N))
    srcb = jnp.concatenate([src, pad_src]).reshape(NBLK_PAD, BLK)
    dstb = jnp.concatenate([dst, pad_dst]).reshape(NBLK_PAD, BLK)

    xl, xr = _mm1(x, W1l, W1r, b1.reshape(1, H))
    acc1, deg = _make_agg(True)(xl, srcb, dstb)
    deg2 = deg[:, :N, None]
    h = _mean_relu(acc1[:, :N], deg2, xr)
    (acc2,) = _make_agg(False)(h, srcb, dstb)
    return _final(acc2[:, :N], deg2, h, W2l, W2r, b2.reshape(1, C))


# speedup vs baseline: 15.7159x; 1.1562x over previous
import functools

import jax
import jax.numpy as jnp
from jax import lax
from jax.experimental import pallas as pl
from jax.experimental.pallas import tpu as pltpu
from jax.experimental.pallas import tpu_sc as plsc

N = 10000
F_IN = 128
H = 16
C = 40
E = 320000

NC = 2
NS = 16
LANES = 16
NW = NC * NS

BLK = 128
NBLK = E // BLK
RB = 16
NCHUNK = -(-NBLK // (NW * RB))
BPW = RB * NCHUNK
NBLK_PAD = BPW * NW
E_PAD = NBLK_PAD * BLK
NPAD = 10240
RPT = NPAD // NS


def _sc_mesh():
    return plsc.VectorSubcoreMesh(
        core_axis_name="c", subcore_axis_name="s",
        num_cores=NC, num_subcores=NS)


def _make_agg(with_deg):
    out_type = [jax.ShapeDtypeStruct((NC, NPAD, H), jnp.float32)]
    scratch = [
        pltpu.VMEM((BPW, BLK), jnp.int32),
        pltpu.VMEM((BPW, BLK), jnp.int32),
        pltpu.VMEM((RB * BLK, H), jnp.float32),
        pltpu.VMEM((RB * BLK, H), jnp.float32),
        pltpu.VMEM((BLK, H), jnp.float32),
        pltpu.VMEM_SHARED((NPAD, H), jnp.float32),
        pltpu.SemaphoreType.DMA,
        pltpu.SemaphoreType.DMA,
    ]
    if with_deg:
        out_type.append(jax.ShapeDtypeStruct((NC, NPAD), jnp.float32))
        scratch += [
            pltpu.VMEM((BLK,), jnp.float32),
            pltpu.VMEM((RPT,), jnp.float32),
            pltpu.VMEM_SHARED((NPAD,), jnp.float32),
        ]

    def body(table, srcb, dstb, *rest):
        if with_deg:
            (out_acc, out_deg, srcv, dstv, rows0, rows1, zrow, acc_sh,
             semg, sems, ones, zdeg, deg_sh) = rest
        else:
            (out_acc, srcv, dstv, rows0, rows1, zrow, acc_sh,
             semg, sems) = rest
        rows = (rows0, rows1)
        cid = lax.axis_index("c")
        sid = lax.axis_index("s")
        wid = cid * NS + sid
        zero16 = jnp.zeros((LANES,), jnp.float32)

        def zrow_body(i, carry):
            zrow[i, :] = zero16
            return carry
        lax.fori_loop(0, BLK, zrow_body, 0)

        row0 = pl.multiple_of(sid * RPT, RPT)
        for z in range(RPT // BLK):
            pltpu.sync_copy(zrow, acc_sh.at[pl.ds(row0 + z * BLK, BLK)])

        if with_deg:
            def zdeg_body(i, carry):
                zdeg[pl.ds(i * LANES, LANES)] = zero16
                return carry
            lax.fori_loop(0, RPT // LANES, zdeg_body, 0)

            one16 = jnp.ones((LANES,), jnp.float32)

            def ones_body(i, carry):
                ones[pl.ds(i * LANES, LANES)] = one16
                return carry
            lax.fori_loop(0, BLK // LANES, ones_body, 0)
            pltpu.sync_copy(zdeg, deg_sh.at[pl.ds(row0, RPT)])

        blk0 = pl.multiple_of(wid * BPW, BPW)
        pltpu.sync_copy(srcb.at[pl.ds(blk0, BPW)], srcv)
        pltpu.sync_copy(dstb.at[pl.ds(blk0, BPW)], dstv)

        plsc.subcore_barrier()

        def fire_g(c):
            buf = rows[c % 2]
            return [
                pltpu.async_copy(table.at[srcv.at[c * RB + k]],
                                 buf.at[pl.ds(k * BLK, BLK)], semg)
                for k in range(RB)
            ]

        def fire_s(c):
            buf = rows[c % 2]
            descs = []
            for k in range(RB):
                descs.append(pltpu.async_copy(
                    buf.at[pl.ds(k * BLK, BLK)],
                    acc_sh.at[dstv.at[c * RB + k]], sems, add=True))
                if with_deg:
                    descs.append(pltpu.async_copy(
                        ones, deg_sh.at[dstv.at[c * RB + k]], sems,
                        add=True))
            return descs

        gd = {0: fire_g(0)}
        sd = {}
        for c in range(NCHUNK):
            for d in gd.pop(c):
                d.wait()
            if c >= 1:
                for d in sd.pop(c - 1):
                    d.wait()
            if c + 1 < NCHUNK:
                gd[c + 1] = fire_g(c + 1)
            sd[c] = fire_s(c)
        for d in sd.pop(NCHUNK - 1):
            d.wait()

        plsc.subcore_barrier()

        pltpu.sync_copy(acc_sh.at[pl.ds(row0, RPT)],
                        out_acc.at[cid, pl.ds(row0, RPT)])
        if with_deg:
            pltpu.sync_copy(deg_sh.at[pl.ds(row0, RPT)],
                            out_deg.at[cid, pl.ds(row0, RPT)])

    return pl.kernel(body, out_type=out_type, mesh=_sc_mesh(),
                     scratch_types=scratch,
                     compiler_params=pltpu.CompilerParams(
                         use_tc_tiling_on_sc=False))


_G = 5
_BM = N // _G


def _mm1(x, W1l, W1r, b1):
    def body(x_ref, wl_ref, wr_ref, b_ref, xl_ref, xr_ref):
        xv = x_ref[...]
        xl_ref[...] = jnp.dot(xv, wl_ref[...],
                              preferred_element_type=jnp.float32)
        xr_ref[...] = jnp.dot(xv, wr_ref[...],
                              preferred_element_type=jnp.float32) + b_ref[...]

    return pl.pallas_call(
        body,
        grid=(_G,),
        in_specs=[
            pl.BlockSpec((_BM, F_IN), lambda i: (i, 0)),
            pl.BlockSpec((F_IN, H), lambda i: (0, 0)),
            pl.BlockSpec((F_IN, H), lambda i: (0, 0)),
            pl.BlockSpec((1, H), lambda i: (0, 0)),
        ],
        out_specs=[
            pl.BlockSpec((_BM, H), lambda i: (i, 0)),
            pl.BlockSpec((_BM, H), lambda i: (i, 0)),
        ],
        out_shape=[
            jax.ShapeDtypeStruct((N, H), jnp.float32),
            jax.ShapeDtypeStruct((N, H), jnp.float32),
        ],
    )(x, W1l, W1r, b1)


def _mean_relu(p, deg2, xr):
    def body(p_ref, d_ref, xr_ref, h_ref):
        s = p_ref[0] + p_ref[1]
        r = 1.0 / jnp.maximum(d_ref[0] + d_ref[1], 1.0)
        h_ref[...] = jnp.maximum(s * r + xr_ref[...], 0.0)

    return pl.pallas_call(
        body,
        grid=(_G,),
        in_specs=[
            pl.BlockSpec((NC, _BM, H), lambda i: (0, i, 0)),
            pl.BlockSpec((NC, _BM, 1), lambda i: (0, i, 0)),
            pl.BlockSpec((_BM, H), lambda i: (i, 0)),
        ],
        out_specs=pl.BlockSpec((_BM, H), lambda i: (i, 0)),
        out_shape=jax.ShapeDtypeStruct((N, H), jnp.float32),
    )(p, deg2, xr)


def _final(q, deg2, h, W2l, W2r, b2):
    def body(q_ref, d_ref, h_ref, wl_ref, wr_ref, b_ref, o_ref):
        m = (q_ref[0] + q_ref[1]) * (
            1.0 / jnp.maximum(d_ref[0] + d_ref[1], 1.0))
        z = (jnp.dot(m, wl_ref[...], preferred_element_type=jnp.float32)
             + jnp.dot(h_ref[...], wr_ref[...],
                       preferred_element_type=jnp.float32)
             + b_ref[...])
        z = z - jnp.max(z, axis=1, keepdims=True)
        o_ref[...] = z - jnp.log(jnp.sum(jnp.exp(z), axis=1, keepdims=True))

    return pl.pallas_call(
        body,
        grid=(_G,),
        in_specs=[
            pl.BlockSpec((NC, _BM, H), lambda i: (0, i, 0)),
            pl.BlockSpec((NC, _BM, 1), lambda i: (0, i, 0)),
            pl.BlockSpec((_BM, H), lambda i: (i, 0)),
            pl.BlockSpec((H, C), lambda i: (0, 0)),
            pl.BlockSpec((H, C), lambda i: (0, 0)),
            pl.BlockSpec((1, C), lambda i: (0, 0)),
        ],
        out_specs=pl.BlockSpec((_BM, C), lambda i: (i, 0)),
        out_shape=jax.ShapeDtypeStruct((N, C), jnp.float32),
    )(q, deg2, h, W2l, W2r, b2)


def kernel(x, edge_index, W1l, W1r, b1, W2l, W2r, b2):
    src = edge_index[0]
    dst = edge_index[1]
    pad_e = E_PAD - E
    pad_src = jnp.zeros((pad_e,), jnp.int32)
    pad_dst = N + (jnp.arange(pad_e, dtype=jnp.int32) % (NPAD - N))
    srcb = jnp.concatenate([src, pad_src]).reshape(NBLK_PAD, BLK)
    dstb = jnp.concatenate([dst, pad_dst]).reshape(NBLK_PAD, BLK)

    xl, xr = _mm1(x, W1l, W1r, b1.reshape(1, H))
    acc1, deg = _make_agg(True)(xl, srcb, dstb)
    deg2 = deg[:, :N, None]
    h = _mean_relu(acc1[:, :N], deg2, xr)
    (acc2,) = _make_agg(False)(h, srcb, dstb)
    return _final(acc2[:, :N], deg2, h, W2l, W2r, b2.reshape(1, C))


# speedup vs baseline: 15.8896x; 1.0111x over previous
import functools

import jax
import jax.numpy as jnp
from jax import lax
from jax.experimental import pallas as pl
from jax.experimental.pallas import tpu as pltpu
from jax.experimental.pallas import tpu_sc as plsc

N = 10000
F_IN = 128
H = 16
C = 40
E = 320000

NC = 2
NS = 16
LANES = 16
NW = NC * NS

BLK = 128
CH = 2048
NCHUNK = -(-E // (NW * CH))
E_PAD = NW * NCHUNK * CH
NPAD = 10240
RPT = NPAD // NS


def _sc_mesh():
    return plsc.VectorSubcoreMesh(
        core_axis_name="c", subcore_axis_name="s",
        num_cores=NC, num_subcores=NS)


def _make_agg(with_deg):
    out_type = [jax.ShapeDtypeStruct((NC, NPAD, H), jnp.float32)]
    scratch = [
        pltpu.VMEM((NCHUNK, CH), jnp.int32),
        pltpu.VMEM((NCHUNK, CH), jnp.int32),
        pltpu.VMEM((CH, H), jnp.float32),
        pltpu.VMEM((CH, H), jnp.float32),
        pltpu.VMEM((BLK, H), jnp.float32),
        pltpu.VMEM_SHARED((NPAD, H), jnp.float32),
        pltpu.SemaphoreType.DMA,
        pltpu.SemaphoreType.DMA,
    ]
    if with_deg:
        out_type.append(jax.ShapeDtypeStruct((NC, NPAD), jnp.float32))
        scratch += [
            pltpu.VMEM((CH,), jnp.float32),
            pltpu.VMEM((RPT,), jnp.float32),
            pltpu.VMEM_SHARED((NPAD,), jnp.float32),
        ]

    def body(table, srcb, dstb, *rest):
        if with_deg:
            (out_acc, out_deg, srcv, dstv, rows0, rows1, zrow, acc_sh,
             semg, sems, ones, zdeg, deg_sh) = rest
        else:
            (out_acc, srcv, dstv, rows0, rows1, zrow, acc_sh,
             semg, sems) = rest
        rows = (rows0, rows1)
        cid = lax.axis_index("c")
        sid = lax.axis_index("s")
        wid = cid * NS + sid
        zero16 = jnp.zeros((LANES,), jnp.float32)

        def zrow_body(i, carry):
            zrow[i, :] = zero16
            return carry
        lax.fori_loop(0, BLK, zrow_body, 0)

        row0 = pl.multiple_of(sid * RPT, RPT)
        for z in range(RPT // BLK):
            pltpu.sync_copy(zrow, acc_sh.at[pl.ds(row0 + z * BLK, BLK)])

        if with_deg:
            def zdeg_body(i, carry):
                zdeg[pl.ds(i * LANES, LANES)] = zero16
                return carry
            lax.fori_loop(0, RPT // LANES, zdeg_body, 0)

            one16 = jnp.ones((LANES,), jnp.float32)

            def ones_body(i, carry):
                ones[pl.ds(i * LANES, LANES)] = one16
                return carry
            lax.fori_loop(0, CH // LANES, ones_body, 0)
            pltpu.sync_copy(zdeg, deg_sh.at[pl.ds(row0, RPT)])

        ch0 = pl.multiple_of(wid * NCHUNK, NCHUNK)
        pltpu.sync_copy(srcb.at[pl.ds(ch0, NCHUNK)], srcv)
        pltpu.sync_copy(dstb.at[pl.ds(ch0, NCHUNK)], dstv)

        plsc.subcore_barrier()

        def fire_g(c):
            return [pltpu.async_copy(table.at[srcv.at[c]], rows[c % 2],
                                     semg)]

        def fire_s(c):
            descs = [pltpu.async_copy(rows[c % 2], acc_sh.at[dstv.at[c]],
                                      sems, add=True)]
            if with_deg:
                descs.append(pltpu.async_copy(ones, deg_sh.at[dstv.at[c]],
                                              sems, add=True))
            return descs

        gd = {0: fire_g(0)}
        sd = {}
        for c in range(NCHUNK):
            for d in gd.pop(c):
                d.wait()
            if c >= 1:
                for d in sd.pop(c - 1):
                    d.wait()
            if c + 1 < NCHUNK:
                gd[c + 1] = fire_g(c + 1)
            sd[c] = fire_s(c)
        for d in sd.pop(NCHUNK - 1):
            d.wait()

        plsc.subcore_barrier()

        pltpu.sync_copy(acc_sh.at[pl.ds(row0, RPT)],
                        out_acc.at[cid, pl.ds(row0, RPT)])
        if with_deg:
            pltpu.sync_copy(deg_sh.at[pl.ds(row0, RPT)],
                            out_deg.at[cid, pl.ds(row0, RPT)])

    return pl.kernel(body, out_type=out_type, mesh=_sc_mesh(),
                     scratch_types=scratch,
                     compiler_params=pltpu.CompilerParams(
                         use_tc_tiling_on_sc=False))


_G = 5
_BM = N // _G


def _mm1(x, W1l, W1r, b1):
    def body(x_ref, wl_ref, wr_ref, b_ref, xl_ref, xr_ref):
        xv = x_ref[...]
        xl_ref[...] = jnp.dot(xv, wl_ref[...],
                              preferred_element_type=jnp.float32)
        xr_ref[...] = jnp.dot(xv, wr_ref[...],
                              preferred_element_type=jnp.float32) + b_ref[...]

    return pl.pallas_call(
        body,
        grid=(_G,),
        in_specs=[
            pl.BlockSpec((_BM, F_IN), lambda i: (i, 0)),
            pl.BlockSpec((F_IN, H), lambda i: (0, 0)),
            pl.BlockSpec((F_IN, H), lambda i: (0, 0)),
            pl.BlockSpec((1, H), lambda i: (0, 0)),
        ],
        out_specs=[
            pl.BlockSpec((_BM, H), lambda i: (i, 0)),
            pl.BlockSpec((_BM, H), lambda i: (i, 0)),
        ],
        out_shape=[
            jax.ShapeDtypeStruct((N, H), jnp.float32),
            jax.ShapeDtypeStruct((N, H), jnp.float32),
        ],
    )(x, W1l, W1r, b1)


def _mean_relu(p, deg2, xr):
    def body(p_ref, d_ref, xr_ref, h_ref):
        s = p_ref[0] + p_ref[1]
        r = 1.0 / jnp.maximum(d_ref[0] + d_ref[1], 1.0)
        h_ref[...] = jnp.maximum(s * r + xr_ref[...], 0.0)

    return pl.pallas_call(
        body,
        grid=(_G,),
        in_specs=[
            pl.BlockSpec((NC, _BM, H), lambda i: (0, i, 0)),
            pl.BlockSpec((NC, _BM, 1), lambda i: (0, i, 0)),
            pl.BlockSpec((_BM, H), lambda i: (i, 0)),
        ],
        out_specs=pl.BlockSpec((_BM, H), lambda i: (i, 0)),
        out_shape=jax.ShapeDtypeStruct((N, H), jnp.float32),
    )(p, deg2, xr)


def _final(q, deg2, h, W2l, W2r, b2):
    def body(q_ref, d_ref, h_ref, wl_ref, wr_ref, b_ref, o_ref):
        m = (q_ref[0] + q_ref[1]) * (
            1.0 / jnp.maximum(d_ref[0] + d_ref[1], 1.0))
        z = (jnp.dot(m, wl_ref[...], preferred_element_type=jnp.float32)
             + jnp.dot(h_ref[...], wr_ref[...],
                       preferred_element_type=jnp.float32)
             + b_ref[...])
        z = z - jnp.max(z, axis=1, keepdims=True)
        o_ref[...] = z - jnp.log(jnp.sum(jnp.exp(z), axis=1, keepdims=True))

    return pl.pallas_call(
        body,
        grid=(_G,),
        in_specs=[
            pl.BlockSpec((NC, _BM, H), lambda i: (0, i, 0)),
            pl.BlockSpec((NC, _BM, 1), lambda i: (0, i, 0)),
            pl.BlockSpec((_BM, H), lambda i: (i, 0)),
            pl.BlockSpec((H, C), lambda i: (0, 0)),
            pl.BlockSpec((H, C), lambda i: (0, 0)),
            pl.BlockSpec((1, C), lambda i: (0, 0)),
        ],
        out_specs=pl.BlockSpec((_BM, C), lambda i: (i, 0)),
        out_shape=jax.ShapeDtypeStruct((N, C), jnp.float32),
    )(q, deg2, h, W2l, W2r, b2)


def kernel(x, edge_index, W1l, W1r, b1, W2l, W2r, b2):
    src = edge_index[0]
    dst = edge_index[1]
    pad_e = E_PAD - E
    pad_src = jnp.zeros((pad_e,), jnp.int32)
    pad_dst = N + (jnp.arange(pad_e, dtype=jnp.int32) % (NPAD - N))
    srcb = jnp.concatenate([src, pad_src]).reshape(NW * NCHUNK, CH)
    dstb = jnp.concatenate([dst, pad_dst]).reshape(NW * NCHUNK, CH)

    xl, xr = _mm1(x, W1l, W1r, b1.reshape(1, H))
    acc1, deg = _make_agg(True)(xl, srcb, dstb)
    deg2 = deg[:, :N, None]
    h = _mean_relu(acc1[:, :N], deg2, xr)
    (acc2,) = _make_agg(False)(h, srcb, dstb)
    return _final(acc2[:, :N], deg2, h, W2l, W2r, b2.reshape(1, C))
